# Initial kernel scaffold; baseline (speedup 1.0000x reference)
import jax
import jax.numpy as jnp
from jax.experimental import pallas as pl


def kernel(x, edge_index, W1_self, W1_neigh, b1, W2_self, W2_neigh, b2, W3_self, W3_neigh, b3):
    raise NotImplementedError("write your pallas kernel here")


# speedup vs baseline: 6.3701x; 6.3701x over previous
import functools

import jax
import jax.numpy as jnp
from jax import lax
from jax.experimental import pallas as pl
from jax.experimental.pallas import tpu as pltpu
from jax.experimental.pallas import tpu_sc as plsc

N_NODES_C = 10000
D_C = 128
N_EDGES_C = 320000
NPAD = 10240

NC = 2
NS = 16
NW = NC * NS
EPW = N_EDGES_C // NW
CHUNK = 80
NCHUNK = EPW // CHUNK
NGRP = 5
GCHUNK = NCHUNK // NGRP
ROWS_PT = NPAD // NS
NSTEP = ROWS_PT // CHUNK


def _sc_agg(h, src, dst, zrows):
    mesh = plsc.VectorSubcoreMesh(core_axis_name="c", subcore_axis_name="s")

    @functools.partial(
        pl.kernel, mesh=mesh,
        out_type=(jax.ShapeDtypeStruct((NC * NPAD, D_C), jnp.float32),),
        scratch_types=[
            pltpu.VMEM((GCHUNK, CHUNK), jnp.int32),
            pltpu.VMEM((GCHUNK, CHUNK), jnp.int32),
            pltpu.VMEM((CHUNK, D_C), jnp.float32),
            pltpu.VMEM_SHARED((NPAD, D_C), jnp.float32),
            pltpu.SemaphoreType.DMA,
        ],
    )
    def k(h_hbm, src_hbm, dst_hbm, z_hbm, out_hbm, sidx, didx, rows, acc, sem):
        c = lax.axis_index("c")
        s = lax.axis_index("s")
        wid = c * NS + s
        base = s * ROWS_PT

        pltpu.sync_copy(z_hbm, rows)

        def zinit(i, carry):
            pltpu.sync_copy(rows, acc.at[pl.ds(base + i * CHUNK, CHUNK)])
            return carry

        lax.fori_loop(0, NSTEP, zinit, 0)
        plsc.subcore_barrier()

        def group(g, carry):
            blk = wid * NGRP + g
            pltpu.sync_copy(src_hbm.at[blk], sidx)
            pltpu.sync_copy(dst_hbm.at[blk], didx)

            def body(j, carry2):
                pltpu.async_copy(h_hbm.at[sidx.at[j]], rows, sem).wait()
                pltpu.sync_copy(rows, acc.at[didx.at[j]], add=True)
                return carry2

            return lax.fori_loop(0, GCHUNK, body, carry)

        lax.fori_loop(0, NGRP, group, 0)
        plsc.subcore_barrier()

        obase = c * NPAD + base

        def drain(i, carry):
            pltpu.sync_copy(acc.at[pl.ds(base + i * CHUNK, CHUNK)], rows)
            pltpu.sync_copy(rows, out_hbm.at[pl.ds(obase + i * CHUNK, CHUNK)])
            return carry

        lax.fori_loop(0, NSTEP, drain, 0)

    return k(h, src, dst, zrows)


def _sc_deg(dst, zrows, ones):
    mesh = plsc.VectorSubcoreMesh(core_axis_name="c", subcore_axis_name="s")

    @functools.partial(
        pl.kernel, mesh=mesh,
        out_type=(jax.ShapeDtypeStruct((NC * NPAD, D_C), jnp.float32),),
        scratch_types=[
            pltpu.VMEM((GCHUNK, CHUNK), jnp.int32),
            pltpu.VMEM((CHUNK, D_C), jnp.float32),
            pltpu.VMEM((CHUNK, D_C), jnp.float32),
            pltpu.VMEM_SHARED((NPAD, D_C), jnp.float32),
        ],
    )
    def k(dst_hbm, z_hbm, ones_hbm, out_hbm, didx, stage, onesv, acc):
        c = lax.axis_index("c")
        s = lax.axis_index("s")
        wid = c * NS + s
        base = s * ROWS_PT
        pltpu.sync_copy(z_hbm, stage)
        pltpu.sync_copy(ones_hbm, onesv)

        def zinit(i, carry):
            pltpu.sync_copy(stage, acc.at[pl.ds(base + i * CHUNK, CHUNK)])
            return carry

        lax.fori_loop(0, NSTEP, zinit, 0)
        plsc.subcore_barrier()

        def group(g, carry):
            pltpu.sync_copy(dst_hbm.at[wid * NGRP + g], didx)

            def body(j, carry2):
                pltpu.sync_copy(onesv, acc.at[didx.at[j]], add=True)
                return carry2

            return lax.fori_loop(0, GCHUNK, body, carry)

        lax.fori_loop(0, NGRP, group, 0)
        plsc.subcore_barrier()
        obase = c * NPAD + base

        def drain(i, carry):
            pltpu.sync_copy(acc.at[pl.ds(base + i * CHUNK, CHUNK)], stage)
            pltpu.sync_copy(stage, out_hbm.at[pl.ds(obase + i * CHUNK, CHUNK)])
            return carry

        lax.fori_loop(0, NSTEP, drain, 0)

    return k(dst, zrows, ones)


def _tc_layer(h, a0, a1, d0, d1, w_self, w_neigh, b, relu):
    BR = 640

    def body(h_ref, a0_ref, a1_ref, d0_ref, d1_ref, ws_ref, wn_ref, b_ref, o_ref):
        deg = d0_ref[:, :1] + d1_ref[:, :1]
        inv = 1.0 / jnp.maximum(deg, 1.0)
        mean = (a0_ref[...] + a1_ref[...]) * inv
        acc = jnp.dot(h_ref[...], ws_ref[...], preferred_element_type=jnp.float32)
        acc = acc + jnp.dot(mean, wn_ref[...], preferred_element_type=jnp.float32)
        acc = acc + b_ref[...]
        o_ref[...] = jnp.maximum(acc, 0.0) if relu else acc

    grid = (NPAD // BR,)
    return pl.pallas_call(
        body,
        grid=grid,
        in_specs=[
            pl.BlockSpec((BR, D_C), lambda i: (i, 0)),
            pl.BlockSpec((BR, D_C), lambda i: (i, 0)),
            pl.BlockSpec((BR, D_C), lambda i: (i, 0)),
            pl.BlockSpec((BR, D_C), lambda i: (i, 0)),
            pl.BlockSpec((BR, D_C), lambda i: (i, 0)),
            pl.BlockSpec((D_C, D_C), lambda i: (0, 0)),
            pl.BlockSpec((D_C, D_C), lambda i: (0, 0)),
            pl.BlockSpec((1, D_C), lambda i: (0, 0)),
        ],
        out_specs=pl.BlockSpec((BR, D_C), lambda i: (i, 0)),
        out_shape=jax.ShapeDtypeStruct((NPAD, D_C), jnp.float32),
    )(h, a0, a1, d0, d1, w_self, w_neigh, b.reshape(1, D_C))


def kernel(x, edge_index, W1_self, W1_neigh, b1, W2_self, W2_neigh, b2,
           W3_self, W3_neigh, b3):
    src = edge_index[0].astype(jnp.int32).reshape(NW * NGRP, GCHUNK, CHUNK)
    dst = edge_index[1].astype(jnp.int32).reshape(NW * NGRP, GCHUNK, CHUNK)
    zrows = jnp.zeros((CHUNK, D_C), jnp.float32)
    ones = jnp.ones((CHUNK, D_C), jnp.float32)
    xp = jnp.pad(x, ((0, NPAD - N_NODES_C), (0, 0)))

    (deg,) = _sc_deg(dst, zrows, ones)
    deg = deg.reshape(NC, NPAD, D_C)
    d0, d1 = deg[0], deg[1]

    (agg1,) = _sc_agg(xp, src, dst, zrows)
    agg1 = agg1.reshape(NC, NPAD, D_C)
    h1 = _tc_layer(xp, agg1[0], agg1[1], d0, d1, W1_self, W1_neigh, b1, relu=True)
    (agg2,) = _sc_agg(h1, src, dst, zrows)
    agg2 = agg2.reshape(NC, NPAD, D_C)
    h2 = _tc_layer(h1, agg2[0], agg2[1], d0, d1, W2_self, W2_neigh, b2, relu=True)
    (agg3,) = _sc_agg(h2, src, dst, zrows)
    agg3 = agg3.reshape(NC, NPAD, D_C)
    h3 = _tc_layer(h2, agg3[0], agg3[1], d0, d1, W3_self, W3_neigh, b3, relu=False)
    return h3[:N_NODES_C]


# speedup vs baseline: 9.0259x; 1.4169x over previous
import functools

import jax
import jax.numpy as jnp
from jax import lax
from jax.experimental import pallas as pl
from jax.experimental.pallas import tpu as pltpu
from jax.experimental.pallas import tpu_sc as plsc

N_NODES_C = 10000
D_C = 128
N_EDGES_C = 320000
NPAD = 10240

NC = 2
NS = 16
NW = NC * NS
EPW = N_EDGES_C // NW
CHUNK = 80
NCHUNK = EPW // CHUNK
NGRP = 5
GCHUNK = NCHUNK // NGRP
ROWS_PT = NPAD // NS
NSTEP = ROWS_PT // CHUNK


def _sc_agg(h, src, dst, zrows):
    mesh = plsc.VectorSubcoreMesh(core_axis_name="c", subcore_axis_name="s")

    @functools.partial(
        pl.kernel, mesh=mesh,
        out_type=(jax.ShapeDtypeStruct((NC * NPAD, D_C), jnp.float32),),
        scratch_types=[
            pltpu.VMEM((GCHUNK, CHUNK), jnp.int32),
            pltpu.VMEM((GCHUNK, CHUNK), jnp.int32),
            pltpu.VMEM((CHUNK, D_C), jnp.float32),
            pltpu.VMEM((CHUNK, D_C), jnp.float32),
            pltpu.VMEM_SHARED((NPAD, D_C), jnp.float32),
            pltpu.SemaphoreType.DMA,
            pltpu.SemaphoreType.DMA,
        ],
    )
    def k(h_hbm, src_hbm, dst_hbm, z_hbm, out_hbm, sidx, didx, rows, rows1,
          acc, sem, sem1):
        c = lax.axis_index("c")
        s = lax.axis_index("s")
        wid = c * NS + s
        base = s * ROWS_PT

        pltpu.sync_copy(z_hbm, rows)

        def zinit(i, carry):
            pltpu.sync_copy(rows, acc.at[pl.ds(base + i * CHUNK, CHUNK)])
            return carry

        lax.fori_loop(0, NSTEP, zinit, 0)
        plsc.subcore_barrier()

        bufs = (rows, rows1)
        sems = (sem, sem1)

        def group(g, carry):
            blk = wid * NGRP + g
            pltpu.sync_copy(src_hbm.at[blk], sidx)
            pltpu.sync_copy(dst_hbm.at[blk], didx)

            pend = [None, None]
            pend[0] = pltpu.async_copy(h_hbm.at[sidx.at[0]], bufs[0], sems[0])
            for j in range(GCHUNK):
                p = j % 2
                if j + 1 < GCHUNK:
                    pend[1 - p] = pltpu.async_copy(
                        h_hbm.at[sidx.at[j + 1]], bufs[1 - p], sems[1 - p])
                pend[p].wait()
                pltpu.sync_copy(bufs[p], acc.at[didx.at[j]], add=True)
            return carry

        lax.fori_loop(0, NGRP, group, 0)
        plsc.subcore_barrier()

        obase = c * NPAD + base

        def drain(i, carry):
            pltpu.sync_copy(acc.at[pl.ds(base + i * CHUNK, CHUNK)], rows)
            pltpu.sync_copy(rows, out_hbm.at[pl.ds(obase + i * CHUNK, CHUNK)])
            return carry

        lax.fori_loop(0, NSTEP, drain, 0)

    return k(h, src, dst, zrows)


def _sc_deg(dst, zrows, ones):
    mesh = plsc.VectorSubcoreMesh(core_axis_name="c", subcore_axis_name="s")

    @functools.partial(
        pl.kernel, mesh=mesh,
        out_type=(jax.ShapeDtypeStruct((NC * NPAD, D_C), jnp.float32),),
        scratch_types=[
            pltpu.VMEM((GCHUNK, CHUNK), jnp.int32),
            pltpu.VMEM((CHUNK, D_C), jnp.float32),
            pltpu.VMEM((CHUNK, D_C), jnp.float32),
            pltpu.VMEM_SHARED((NPAD, D_C), jnp.float32),
        ],
    )
    def k(dst_hbm, z_hbm, ones_hbm, out_hbm, didx, stage, onesv, acc):
        c = lax.axis_index("c")
        s = lax.axis_index("s")
        wid = c * NS + s
        base = s * ROWS_PT
        pltpu.sync_copy(z_hbm, stage)
        pltpu.sync_copy(ones_hbm, onesv)

        def zinit(i, carry):
            pltpu.sync_copy(stage, acc.at[pl.ds(base + i * CHUNK, CHUNK)])
            return carry

        lax.fori_loop(0, NSTEP, zinit, 0)
        plsc.subcore_barrier()

        def group(g, carry):
            pltpu.sync_copy(dst_hbm.at[wid * NGRP + g], didx)

            def body(j, carry2):
                pltpu.sync_copy(onesv, acc.at[didx.at[j]], add=True)
                return carry2

            return lax.fori_loop(0, GCHUNK, body, carry)

        lax.fori_loop(0, NGRP, group, 0)
        plsc.subcore_barrier()
        obase = c * NPAD + base

        def drain(i, carry):
            pltpu.sync_copy(acc.at[pl.ds(base + i * CHUNK, CHUNK)], stage)
            pltpu.sync_copy(stage, out_hbm.at[pl.ds(obase + i * CHUNK, CHUNK)])
            return carry

        lax.fori_loop(0, NSTEP, drain, 0)

    return k(dst, zrows, ones)


def _tc_layer(h, a0, a1, d0, d1, w_self, w_neigh, b, relu):
    BR = 640

    def body(h_ref, a0_ref, a1_ref, d0_ref, d1_ref, ws_ref, wn_ref, b_ref, o_ref):
        deg = d0_ref[:, :1] + d1_ref[:, :1]
        inv = 1.0 / jnp.maximum(deg, 1.0)
        mean = (a0_ref[...] + a1_ref[...]) * inv
        acc = jnp.dot(h_ref[...], ws_ref[...], preferred_element_type=jnp.float32)
        acc = acc + jnp.dot(mean, wn_ref[...], preferred_element_type=jnp.float32)
        acc = acc + b_ref[...]
        o_ref[...] = jnp.maximum(acc, 0.0) if relu else acc

    grid = (NPAD // BR,)
    return pl.pallas_call(
        body,
        grid=grid,
        in_specs=[
            pl.BlockSpec((BR, D_C), lambda i: (i, 0)),
            pl.BlockSpec((BR, D_C), lambda i: (i, 0)),
            pl.BlockSpec((BR, D_C), lambda i: (i, 0)),
            pl.BlockSpec((BR, D_C), lambda i: (i, 0)),
            pl.BlockSpec((BR, D_C), lambda i: (i, 0)),
            pl.BlockSpec((D_C, D_C), lambda i: (0, 0)),
            pl.BlockSpec((D_C, D_C), lambda i: (0, 0)),
            pl.BlockSpec((1, D_C), lambda i: (0, 0)),
        ],
        out_specs=pl.BlockSpec((BR, D_C), lambda i: (i, 0)),
        out_shape=jax.ShapeDtypeStruct((NPAD, D_C), jnp.float32),
    )(h, a0, a1, d0, d1, w_self, w_neigh, b.reshape(1, D_C))


def kernel(x, edge_index, W1_self, W1_neigh, b1, W2_self, W2_neigh, b2,
           W3_self, W3_neigh, b3):
    src = edge_index[0].astype(jnp.int32).reshape(NW * NGRP, GCHUNK, CHUNK)
    dst = edge_index[1].astype(jnp.int32).reshape(NW * NGRP, GCHUNK, CHUNK)
    zrows = jnp.zeros((CHUNK, D_C), jnp.float32)
    ones = jnp.ones((CHUNK, D_C), jnp.float32)
    xp = jnp.pad(x, ((0, NPAD - N_NODES_C), (0, 0)))

    (deg,) = _sc_deg(dst, zrows, ones)
    deg = deg.reshape(NC, NPAD, D_C)
    d0, d1 = deg[0], deg[1]

    (agg1,) = _sc_agg(xp, src, dst, zrows)
    agg1 = agg1.reshape(NC, NPAD, D_C)
    h1 = _tc_layer(xp, agg1[0], agg1[1], d0, d1, W1_self, W1_neigh, b1, relu=True)
    (agg2,) = _sc_agg(h1, src, dst, zrows)
    agg2 = agg2.reshape(NC, NPAD, D_C)
    h2 = _tc_layer(h1, agg2[0], agg2[1], d0, d1, W2_self, W2_neigh, b2, relu=True)
    (agg3,) = _sc_agg(h2, src, dst, zrows)
    agg3 = agg3.reshape(NC, NPAD, D_C)
    h3 = _tc_layer(h2, agg3[0], agg3[1], d0, d1, W3_self, W3_neigh, b3, relu=False)
    return h3[:N_NODES_C]


# speedup vs baseline: 10.0936x; 1.1183x over previous
import functools

import jax
import jax.numpy as jnp
from jax import lax
from jax.experimental import pallas as pl
from jax.experimental.pallas import tpu as pltpu
from jax.experimental.pallas import tpu_sc as plsc

N_NODES_C = 10000
D_C = 128
N_EDGES_C = 320000
NPAD = 10240

NC = 2
NS = 16
NW = NC * NS
EPW = N_EDGES_C // NW
CHUNK = 80
NCHUNK = EPW // CHUNK
NGRP = 5
GCHUNK = NCHUNK // NGRP
ROWS_PT = NPAD // NS
NSTEP = ROWS_PT // CHUNK


def _sc_agg(h, src, dst, zrows):
    mesh = plsc.VectorSubcoreMesh(core_axis_name="c", subcore_axis_name="s")

    @functools.partial(
        pl.kernel, mesh=mesh,
        out_type=(jax.ShapeDtypeStruct((NC * NPAD, D_C), jnp.float32),),
        scratch_types=[
            pltpu.VMEM((GCHUNK, CHUNK), jnp.int32),
            pltpu.VMEM((GCHUNK, CHUNK), jnp.int32),
            pltpu.VMEM((CHUNK, D_C), jnp.float32),
            pltpu.VMEM((CHUNK, D_C), jnp.float32),
            pltpu.VMEM((CHUNK, D_C), jnp.float32),
            pltpu.VMEM((CHUNK, D_C), jnp.float32),
            pltpu.VMEM_SHARED((NPAD, D_C), jnp.float32),
            pltpu.SemaphoreType.DMA,
            pltpu.SemaphoreType.DMA,
            pltpu.SemaphoreType.DMA,
            pltpu.SemaphoreType.DMA,
            pltpu.SemaphoreType.DMA,
            pltpu.SemaphoreType.DMA,
            pltpu.SemaphoreType.DMA,
            pltpu.SemaphoreType.DMA,
        ],
    )
    def k(h_hbm, src_hbm, dst_hbm, z_hbm, out_hbm, sidx, didx, rows, rows1,
          rows2, rows3, acc, sem, sem1, sem2, sem3, ssem, ssem1, ssem2, ssem3):
        c = lax.axis_index("c")
        s = lax.axis_index("s")
        wid = c * NS + s
        base = s * ROWS_PT

        pltpu.sync_copy(z_hbm, rows)

        def zinit(i, carry):
            pltpu.sync_copy(rows, acc.at[pl.ds(base + i * CHUNK, CHUNK)])
            return carry

        lax.fori_loop(0, NSTEP, zinit, 0)
        plsc.subcore_barrier()

        bufs = (rows, rows1, rows2, rows3)
        gsems = (sem, sem1, sem2, sem3)
        ssems = (ssem, ssem1, ssem2, ssem3)
        NBUF = 4

        def group(g, carry):
            blk = wid * NGRP + g
            pltpu.sync_copy(src_hbm.at[blk], sidx)
            pltpu.sync_copy(dst_hbm.at[blk], didx)

            gpend = [None] * NBUF
            spend = [None] * NBUF
            for j in range(NBUF - 1):
                gpend[j] = pltpu.async_copy(
                    h_hbm.at[sidx.at[j]], bufs[j], gsems[j])
            for j in range(GCHUNK):
                p = j % NBUF
                nxt = j + NBUF - 1
                if nxt < GCHUNK:
                    q = nxt % NBUF
                    if spend[q] is not None:
                        spend[q].wait()
                    gpend[q] = pltpu.async_copy(
                        h_hbm.at[sidx.at[nxt]], bufs[q], gsems[q])
                gpend[p].wait()
                spend[p] = pltpu.async_copy(
                    bufs[p], acc.at[didx.at[j]], ssems[p], add=True)
            for p in range(NBUF):
                if spend[p] is not None:
                    spend[p].wait()
            return carry

        lax.fori_loop(0, NGRP, group, 0)
        plsc.subcore_barrier()

        obase = c * NPAD + base

        def drain(i, carry):
            pltpu.sync_copy(acc.at[pl.ds(base + i * CHUNK, CHUNK)], rows)
            pltpu.sync_copy(rows, out_hbm.at[pl.ds(obase + i * CHUNK, CHUNK)])
            return carry

        lax.fori_loop(0, NSTEP, drain, 0)

    return k(h, src, dst, zrows)


def _sc_deg(dst, zrows, ones):
    mesh = plsc.VectorSubcoreMesh(core_axis_name="c", subcore_axis_name="s")

    @functools.partial(
        pl.kernel, mesh=mesh,
        out_type=(jax.ShapeDtypeStruct((NC * NPAD, D_C), jnp.float32),),
        scratch_types=[
            pltpu.VMEM((GCHUNK, CHUNK), jnp.int32),
            pltpu.VMEM((CHUNK, D_C), jnp.float32),
            pltpu.VMEM((CHUNK, D_C), jnp.float32),
            pltpu.VMEM_SHARED((NPAD, D_C), jnp.float32),
        ],
    )
    def k(dst_hbm, z_hbm, ones_hbm, out_hbm, didx, stage, onesv, acc):
        c = lax.axis_index("c")
        s = lax.axis_index("s")
        wid = c * NS + s
        base = s * ROWS_PT
        pltpu.sync_copy(z_hbm, stage)
        pltpu.sync_copy(ones_hbm, onesv)

        def zinit(i, carry):
            pltpu.sync_copy(stage, acc.at[pl.ds(base + i * CHUNK, CHUNK)])
            return carry

        lax.fori_loop(0, NSTEP, zinit, 0)
        plsc.subcore_barrier()

        def group(g, carry):
            pltpu.sync_copy(dst_hbm.at[wid * NGRP + g], didx)

            def body(j, carry2):
                pltpu.sync_copy(onesv, acc.at[didx.at[j]], add=True)
                return carry2

            return lax.fori_loop(0, GCHUNK, body, carry)

        lax.fori_loop(0, NGRP, group, 0)
        plsc.subcore_barrier()
        obase = c * NPAD + base

        def drain(i, carry):
            pltpu.sync_copy(acc.at[pl.ds(base + i * CHUNK, CHUNK)], stage)
            pltpu.sync_copy(stage, out_hbm.at[pl.ds(obase + i * CHUNK, CHUNK)])
            return carry

        lax.fori_loop(0, NSTEP, drain, 0)

    return k(dst, zrows, ones)


def _tc_layer(h, a0, a1, d0, d1, w_self, w_neigh, b, relu):
    BR = 640

    def body(h_ref, a0_ref, a1_ref, d0_ref, d1_ref, ws_ref, wn_ref, b_ref, o_ref):
        deg = d0_ref[:, :1] + d1_ref[:, :1]
        inv = 1.0 / jnp.maximum(deg, 1.0)
        mean = (a0_ref[...] + a1_ref[...]) * inv
        acc = jnp.dot(h_ref[...], ws_ref[...], preferred_element_type=jnp.float32)
        acc = acc + jnp.dot(mean, wn_ref[...], preferred_element_type=jnp.float32)
        acc = acc + b_ref[...]
        o_ref[...] = jnp.maximum(acc, 0.0) if relu else acc

    grid = (NPAD // BR,)
    return pl.pallas_call(
        body,
        grid=grid,
        in_specs=[
            pl.BlockSpec((BR, D_C), lambda i: (i, 0)),
            pl.BlockSpec((BR, D_C), lambda i: (i, 0)),
            pl.BlockSpec((BR, D_C), lambda i: (i, 0)),
            pl.BlockSpec((BR, D_C), lambda i: (i, 0)),
            pl.BlockSpec((BR, D_C), lambda i: (i, 0)),
            pl.BlockSpec((D_C, D_C), lambda i: (0, 0)),
            pl.BlockSpec((D_C, D_C), lambda i: (0, 0)),
            pl.BlockSpec((1, D_C), lambda i: (0, 0)),
        ],
        out_specs=pl.BlockSpec((BR, D_C), lambda i: (i, 0)),
        out_shape=jax.ShapeDtypeStruct((NPAD, D_C), jnp.float32),
    )(h, a0, a1, d0, d1, w_self, w_neigh, b.reshape(1, D_C))


def kernel(x, edge_index, W1_self, W1_neigh, b1, W2_self, W2_neigh, b2,
           W3_self, W3_neigh, b3):
    src = edge_index[0].astype(jnp.int32).reshape(NW * NGRP, GCHUNK, CHUNK)
    dst = edge_index[1].astype(jnp.int32).reshape(NW * NGRP, GCHUNK, CHUNK)
    zrows = jnp.zeros((CHUNK, D_C), jnp.float32)
    ones = jnp.ones((CHUNK, D_C), jnp.float32)
    xp = jnp.pad(x, ((0, NPAD - N_NODES_C), (0, 0)))

    (deg,) = _sc_deg(dst, zrows, ones)
    deg = deg.reshape(NC, NPAD, D_C)
    d0, d1 = deg[0], deg[1]

    (agg1,) = _sc_agg(xp, src, dst, zrows)
    agg1 = agg1.reshape(NC, NPAD, D_C)
    h1 = _tc_layer(xp, agg1[0], agg1[1], d0, d1, W1_self, W1_neigh, b1, relu=True)
    (agg2,) = _sc_agg(h1, src, dst, zrows)
    agg2 = agg2.reshape(NC, NPAD, D_C)
    h2 = _tc_layer(h1, agg2[0], agg2[1], d0, d1, W2_self, W2_neigh, b2, relu=True)
    (agg3,) = _sc_agg(h2, src, dst, zrows)
    agg3 = agg3.reshape(NC, NPAD, D_C)
    h3 = _tc_layer(h2, agg3[0], agg3[1], d0, d1, W3_self, W3_neigh, b3, relu=False)
    return h3[:N_NODES_C]


# speedup vs baseline: 10.2716x; 1.0176x over previous
import functools

import jax
import jax.numpy as jnp
from jax import lax
from jax.experimental import pallas as pl
from jax.experimental.pallas import tpu as pltpu
from jax.experimental.pallas import tpu_sc as plsc

N_NODES_C = 10000
D_C = 128
N_EDGES_C = 320000
NPAD = 10240

NC = 2
NS = 16
NW = NC * NS
EPW = N_EDGES_C // NW
CHUNK = 80
NCHUNK = EPW // CHUNK
NGRP = 5
GCHUNK = NCHUNK // NGRP
ROWS_PT = NPAD // NS
NSTEP = ROWS_PT // CHUNK


def _sc_agg(h, src, dst, zrows):
    mesh = plsc.VectorSubcoreMesh(core_axis_name="c", subcore_axis_name="s")

    @functools.partial(
        pl.kernel, mesh=mesh,
        out_type=(jax.ShapeDtypeStruct((NC * NPAD, D_C), jnp.float32),),
        scratch_types=[
            pltpu.VMEM((GCHUNK, CHUNK), jnp.int32),
            pltpu.VMEM((GCHUNK, CHUNK), jnp.int32),
            pltpu.VMEM((CHUNK, D_C), jnp.float32),
            pltpu.VMEM((CHUNK, D_C), jnp.float32),
            pltpu.VMEM((CHUNK, D_C), jnp.float32),
            pltpu.VMEM((CHUNK, D_C), jnp.float32),
            pltpu.VMEM_SHARED((NPAD, D_C), jnp.float32),
            pltpu.SemaphoreType.DMA,
            pltpu.SemaphoreType.DMA,
            pltpu.SemaphoreType.DMA,
            pltpu.SemaphoreType.DMA,
            pltpu.SemaphoreType.DMA,
            pltpu.SemaphoreType.DMA,
            pltpu.SemaphoreType.DMA,
            pltpu.SemaphoreType.DMA,
        ],
    )
    def k(h_hbm, src_hbm, dst_hbm, z_hbm, out_hbm, sidx, didx, rows, rows1,
          rows2, rows3, acc, sem, sem1, sem2, sem3, ssem, ssem1, ssem2, ssem3):
        c = lax.axis_index("c")
        s = lax.axis_index("s")
        wid = c * NS + s
        base = s * ROWS_PT

        pltpu.sync_copy(z_hbm, rows)
        zpend = [
            pltpu.async_copy(rows, acc.at[pl.ds(base + i * CHUNK, CHUNK)],
                             (sem, sem1, sem2, sem3)[i % 4])
            for i in range(NSTEP)
        ]
        for hnd in zpend:
            hnd.wait()
        plsc.subcore_barrier()

        bufs = (rows, rows1, rows2, rows3)
        gsems = (sem, sem1, sem2, sem3)
        ssems = (ssem, ssem1, ssem2, ssem3)
        NBUF = 4

        def group(g, carry):
            blk = wid * NGRP + g
            pltpu.sync_copy(src_hbm.at[blk], sidx)
            pltpu.sync_copy(dst_hbm.at[blk], didx)

            gpend = [None] * NBUF
            spend = [None] * NBUF
            for j in range(NBUF - 1):
                gpend[j] = pltpu.async_copy(
                    h_hbm.at[sidx.at[j]], bufs[j], gsems[j])
            for j in range(GCHUNK):
                p = j % NBUF
                nxt = j + NBUF - 1
                if nxt < GCHUNK:
                    q = nxt % NBUF
                    if spend[q] is not None:
                        spend[q].wait()
                    gpend[q] = pltpu.async_copy(
                        h_hbm.at[sidx.at[nxt]], bufs[q], gsems[q])
                gpend[p].wait()
                spend[p] = pltpu.async_copy(
                    bufs[p], acc.at[didx.at[j]], ssems[p], add=True)
            for p in range(NBUF):
                if spend[p] is not None:
                    spend[p].wait()
            return carry

        lax.fori_loop(0, NGRP, group, 0)
        plsc.subcore_barrier()

        obase = c * NPAD + base
        wpend = [None] * NBUF
        for i in range(NSTEP):
            p = i % NBUF
            if wpend[p] is not None:
                wpend[p].wait()
            pltpu.sync_copy(acc.at[pl.ds(base + i * CHUNK, CHUNK)], bufs[p])
            wpend[p] = pltpu.async_copy(
                bufs[p], out_hbm.at[pl.ds(obase + i * CHUNK, CHUNK)], gsems[p])
        for p in range(NBUF):
            if wpend[p] is not None:
                wpend[p].wait()

    return k(h, src, dst, zrows)


def _sc_deg(dst, zrows, ones):
    mesh = plsc.VectorSubcoreMesh(core_axis_name="c", subcore_axis_name="s")

    @functools.partial(
        pl.kernel, mesh=mesh,
        out_type=(jax.ShapeDtypeStruct((NC * NPAD, D_C), jnp.float32),),
        scratch_types=[
            pltpu.VMEM((GCHUNK, CHUNK), jnp.int32),
            pltpu.VMEM((CHUNK, D_C), jnp.float32),
            pltpu.VMEM((CHUNK, D_C), jnp.float32),
            pltpu.VMEM_SHARED((NPAD, D_C), jnp.float32),
            pltpu.SemaphoreType.DMA,
            pltpu.SemaphoreType.DMA,
            pltpu.SemaphoreType.DMA,
            pltpu.SemaphoreType.DMA,
        ],
    )
    def k(dst_hbm, z_hbm, ones_hbm, out_hbm, didx, stage, onesv, acc,
          dsem, dsem1, dsem2, dsem3):
        c = lax.axis_index("c")
        s = lax.axis_index("s")
        wid = c * NS + s
        base = s * ROWS_PT
        pltpu.sync_copy(z_hbm, stage)
        pltpu.sync_copy(ones_hbm, onesv)

        def zinit(i, carry):
            pltpu.sync_copy(stage, acc.at[pl.ds(base + i * CHUNK, CHUNK)])
            return carry

        lax.fori_loop(0, NSTEP, zinit, 0)
        plsc.subcore_barrier()

        dsems = (dsem, dsem1, dsem2, dsem3)

        def group(g, carry):
            pltpu.sync_copy(dst_hbm.at[wid * NGRP + g], didx)
            dpend = [None] * 4
            for j in range(GCHUNK):
                q = j % 4
                if dpend[q] is not None:
                    dpend[q].wait()
                dpend[q] = pltpu.async_copy(
                    onesv, acc.at[didx.at[j]], dsems[q], add=True)
            for q in range(4):
                if dpend[q] is not None:
                    dpend[q].wait()
            return carry

        lax.fori_loop(0, NGRP, group, 0)
        plsc.subcore_barrier()
        obase = c * NPAD + base

        def drain(i, carry):
            pltpu.sync_copy(acc.at[pl.ds(base + i * CHUNK, CHUNK)], stage)
            pltpu.sync_copy(stage, out_hbm.at[pl.ds(obase + i * CHUNK, CHUNK)])
            return carry

        lax.fori_loop(0, NSTEP, drain, 0)

    return k(dst, zrows, ones)


def _tc_layer(h, a0, a1, d0, d1, w_self, w_neigh, b, relu):
    BR = 640

    def body(h_ref, a0_ref, a1_ref, d0_ref, d1_ref, ws_ref, wn_ref, b_ref, o_ref):
        deg = d0_ref[:, :1] + d1_ref[:, :1]
        inv = 1.0 / jnp.maximum(deg, 1.0)
        mean = (a0_ref[...] + a1_ref[...]) * inv
        acc = jnp.dot(h_ref[...], ws_ref[...], preferred_element_type=jnp.float32)
        acc = acc + jnp.dot(mean, wn_ref[...], preferred_element_type=jnp.float32)
        acc = acc + b_ref[...]
        o_ref[...] = jnp.maximum(acc, 0.0) if relu else acc

    grid = (NPAD // BR,)
    return pl.pallas_call(
        body,
        grid=grid,
        in_specs=[
            pl.BlockSpec((BR, D_C), lambda i: (i, 0)),
            pl.BlockSpec((BR, D_C), lambda i: (i, 0)),
            pl.BlockSpec((BR, D_C), lambda i: (i, 0)),
            pl.BlockSpec((BR, D_C), lambda i: (i, 0)),
            pl.BlockSpec((BR, D_C), lambda i: (i, 0)),
            pl.BlockSpec((D_C, D_C), lambda i: (0, 0)),
            pl.BlockSpec((D_C, D_C), lambda i: (0, 0)),
            pl.BlockSpec((1, D_C), lambda i: (0, 0)),
        ],
        out_specs=pl.BlockSpec((BR, D_C), lambda i: (i, 0)),
        out_shape=jax.ShapeDtypeStruct((NPAD, D_C), jnp.float32),
    )(h, a0, a1, d0, d1, w_self, w_neigh, b.reshape(1, D_C))


def kernel(x, edge_index, W1_self, W1_neigh, b1, W2_self, W2_neigh, b2,
           W3_self, W3_neigh, b3):
    src = edge_index[0].astype(jnp.int32).reshape(NW * NGRP, GCHUNK, CHUNK)
    dst = edge_index[1].astype(jnp.int32).reshape(NW * NGRP, GCHUNK, CHUNK)
    zrows = jnp.zeros((CHUNK, D_C), jnp.float32)
    ones = jnp.ones((CHUNK, D_C), jnp.float32)
    xp = jnp.pad(x, ((0, NPAD - N_NODES_C), (0, 0)))

    (deg,) = _sc_deg(dst, zrows, ones)
    deg = deg.reshape(NC, NPAD, D_C)
    d0, d1 = deg[0], deg[1]

    (agg1,) = _sc_agg(xp, src, dst, zrows)
    agg1 = agg1.reshape(NC, NPAD, D_C)
    h1 = _tc_layer(xp, agg1[0], agg1[1], d0, d1, W1_self, W1_neigh, b1, relu=True)
    (agg2,) = _sc_agg(h1, src, dst, zrows)
    agg2 = agg2.reshape(NC, NPAD, D_C)
    h2 = _tc_layer(h1, agg2[0], agg2[1], d0, d1, W2_self, W2_neigh, b2, relu=True)
    (agg3,) = _sc_agg(h2, src, dst, zrows)
    agg3 = agg3.reshape(NC, NPAD, D_C)
    h3 = _tc_layer(h2, agg3[0], agg3[1], d0, d1, W3_self, W3_neigh, b3, relu=False)
    return h3[:N_NODES_C]


# speedup vs baseline: 11.5195x; 1.1215x over previous
import functools

import jax
import jax.numpy as jnp
from jax import lax
from jax.experimental import pallas as pl
from jax.experimental.pallas import tpu as pltpu
from jax.experimental.pallas import tpu_sc as plsc

N_NODES_C = 10000
D_C = 128
N_EDGES_C = 320000
NPAD = 10240

NC = 2
NS = 16
NW = NC * NS
EPW = N_EDGES_C // NW
CHUNK = 80
NCHUNK = EPW // CHUNK
NGRP = 5
GCHUNK = NCHUNK // NGRP
ROWS_PT = NPAD // NS
NSTEP = ROWS_PT // CHUNK
L = 16


def _sc_agg(h, src, dst, zrows):
    mesh = plsc.VectorSubcoreMesh(core_axis_name="c", subcore_axis_name="s")

    nbuf = 4
    out_type = [jax.ShapeDtypeStruct((NC * NPAD, D_C), jnp.float32)]
    scratch = [
        pltpu.VMEM((GCHUNK, CHUNK), jnp.int32),
        pltpu.VMEM((GCHUNK, CHUNK), jnp.int32),
        pltpu.VMEM_SHARED((NPAD, D_C), jnp.float32),
    ]
    for _ in range(nbuf):
        scratch.append(pltpu.VMEM((CHUNK, D_C), jnp.float32))
    for _ in range(2 * nbuf):
        scratch.append(pltpu.SemaphoreType.DMA)

    @functools.partial(
        pl.kernel, mesh=mesh, out_type=tuple(out_type), scratch_types=scratch,
    )
    def k(h_hbm, src_hbm, dst_hbm, z_hbm, *rest):
        rest = list(rest)
        out_hbm = rest.pop(0)
        sidx = rest.pop(0)
        didx = rest.pop(0)
        acc = rest.pop(0)
        bufs = tuple(rest.pop(0) for _ in range(nbuf))
        gsems = tuple(rest.pop(0) for _ in range(nbuf))
        ssems = tuple(rest.pop(0) for _ in range(nbuf))

        c = lax.axis_index("c")
        s = lax.axis_index("s")
        wid = c * NS + s
        base = s * ROWS_PT

        pltpu.sync_copy(z_hbm, bufs[0])
        zpend = [
            pltpu.async_copy(bufs[0], acc.at[pl.ds(base + i * CHUNK, CHUNK)],
                             gsems[i % nbuf])
            for i in range(NSTEP)
        ]
        for hnd in zpend:
            hnd.wait()
        plsc.subcore_barrier()

        def group(g, carry):
            blk = wid * NGRP + g
            pltpu.sync_copy(src_hbm.at[blk], sidx)
            pltpu.sync_copy(dst_hbm.at[blk], didx)

            gpend = [None] * nbuf
            spend = [None] * nbuf
            for j in range(nbuf - 1):
                gpend[j] = pltpu.async_copy(
                    h_hbm.at[sidx.at[j]], bufs[j], gsems[j])
            for j in range(GCHUNK):
                p = j % nbuf
                nxt = j + nbuf - 1
                if nxt < GCHUNK:
                    q = nxt % nbuf
                    if spend[q] is not None:
                        spend[q].wait()
                    gpend[q] = pltpu.async_copy(
                        h_hbm.at[sidx.at[nxt]], bufs[q], gsems[q])
                gpend[p].wait()
                spend[p] = pltpu.async_copy(
                    bufs[p], acc.at[didx.at[j]], ssems[p], add=True)
            for p in range(nbuf):
                if spend[p] is not None:
                    spend[p].wait()
            return carry

        lax.fori_loop(0, NGRP, group, 0)
        plsc.subcore_barrier()

        obase = c * NPAD + base
        wpend = [None] * nbuf
        for i in range(NSTEP):
            p = i % nbuf
            if wpend[p] is not None:
                wpend[p].wait()
            pltpu.sync_copy(acc.at[pl.ds(base + i * CHUNK, CHUNK)], bufs[p])
            wpend[p] = pltpu.async_copy(
                bufs[p], out_hbm.at[pl.ds(obase + i * CHUNK, CHUNK)], gsems[p])
        for p in range(nbuf):
            if wpend[p] is not None:
                wpend[p].wait()

    return k(h, src, dst, zrows)


def _sc_deg(dst, zrows, ones):
    mesh = plsc.VectorSubcoreMesh(core_axis_name="c", subcore_axis_name="s")

    @functools.partial(
        pl.kernel, mesh=mesh,
        out_type=(jax.ShapeDtypeStruct((NC * NPAD, D_C), jnp.float32),),
        scratch_types=[
            pltpu.VMEM((GCHUNK, CHUNK), jnp.int32),
            pltpu.VMEM((CHUNK, D_C), jnp.float32),
            pltpu.VMEM((CHUNK, D_C), jnp.float32),
            pltpu.VMEM_SHARED((NPAD, D_C), jnp.float32),
            pltpu.SemaphoreType.DMA,
            pltpu.SemaphoreType.DMA,
            pltpu.SemaphoreType.DMA,
            pltpu.SemaphoreType.DMA,
        ],
    )
    def k(dst_hbm, z_hbm, ones_hbm, out_hbm, didx, stage, onesv, acc,
          dsem, dsem1, dsem2, dsem3):
        c = lax.axis_index("c")
        s = lax.axis_index("s")
        wid = c * NS + s
        base = s * ROWS_PT
        dsems = (dsem, dsem1, dsem2, dsem3)
        pltpu.sync_copy(z_hbm, stage)
        pltpu.sync_copy(ones_hbm, onesv)
        zpend = [
            pltpu.async_copy(stage, acc.at[pl.ds(base + i * CHUNK, CHUNK)],
                             dsems[i % 4])
            for i in range(NSTEP)
        ]
        for hnd in zpend:
            hnd.wait()
        plsc.subcore_barrier()

        def group(g, carry):
            pltpu.sync_copy(dst_hbm.at[wid * NGRP + g], didx)
            dpend = [None] * 4
            for j in range(GCHUNK):
                q = j % 4
                if dpend[q] is not None:
                    dpend[q].wait()
                dpend[q] = pltpu.async_copy(
                    onesv, acc.at[didx.at[j]], dsems[q], add=True)
            for q in range(4):
                if dpend[q] is not None:
                    dpend[q].wait()
            return carry

        lax.fori_loop(0, NGRP, group, 0)
        plsc.subcore_barrier()
        obase = c * NPAD + base
        wpend = [None] * 2
        for i in range(NSTEP):
            p = i % 2
            if wpend[p] is not None:
                wpend[p].wait()
            buf = (stage, onesv)[p]
            pltpu.sync_copy(acc.at[pl.ds(base + i * CHUNK, CHUNK)], buf)
            wpend[p] = pltpu.async_copy(
                buf, out_hbm.at[pl.ds(obase + i * CHUNK, CHUNK)], dsems[p])
        for p in range(2):
            if wpend[p] is not None:
                wpend[p].wait()

    return k(dst, zrows, ones)


def _tc_layer(h, agg, deg, w_self, w_neigh, b, relu):
    BR = 2048
    NB = NPAD // BR

    def body(h_ref, a0_ref, a1_ref, d0_ref, d1_ref, ws_ref, wn_ref, b_ref,
             o_ref):
        deg_c = d0_ref[:, :1] + d1_ref[:, :1]
        inv = 1.0 / jnp.maximum(deg_c, 1.0)
        mean = (a0_ref[...] + a1_ref[...]) * inv
        acc = jnp.dot(h_ref[...], ws_ref[...], preferred_element_type=jnp.float32)
        acc = acc + jnp.dot(mean, wn_ref[...], preferred_element_type=jnp.float32)
        acc = acc + b_ref[...]
        o_ref[...] = jnp.maximum(acc, 0.0) if relu else acc

    return pl.pallas_call(
        body,
        grid=(NB,),
        in_specs=[
            pl.BlockSpec((BR, D_C), lambda i: (i, 0)),
            pl.BlockSpec((BR, D_C), lambda i: (i, 0)),
            pl.BlockSpec((BR, D_C), lambda i: (NB + i, 0)),
            pl.BlockSpec((BR, D_C), lambda i: (i, 0)),
            pl.BlockSpec((BR, D_C), lambda i: (NB + i, 0)),
            pl.BlockSpec((D_C, D_C), lambda i: (0, 0)),
            pl.BlockSpec((D_C, D_C), lambda i: (0, 0)),
            pl.BlockSpec((1, D_C), lambda i: (0, 0)),
        ],
        out_specs=pl.BlockSpec((BR, D_C), lambda i: (i, 0)),
        out_shape=jax.ShapeDtypeStruct((NPAD, D_C), jnp.float32),
    )(h, agg, agg, deg, deg, w_self, w_neigh, b.reshape(1, D_C))


def kernel(x, edge_index, W1_self, W1_neigh, b1, W2_self, W2_neigh, b2,
           W3_self, W3_neigh, b3):
    src = edge_index[0].astype(jnp.int32).reshape(NW * NGRP, GCHUNK, CHUNK)
    dst = edge_index[1].astype(jnp.int32).reshape(NW * NGRP, GCHUNK, CHUNK)
    zrows = jnp.zeros((CHUNK, D_C), jnp.float32)
    ones = jnp.ones((CHUNK, D_C), jnp.float32)
    xp = jnp.pad(x, ((0, NPAD - N_NODES_C), (0, 0)))

    (deg,) = _sc_deg(dst, zrows, ones)
    (agg1,) = _sc_agg(xp, src, dst, zrows)
    h1 = _tc_layer(xp, agg1, deg, W1_self, W1_neigh, b1, relu=True)
    (agg2,) = _sc_agg(h1, src, dst, zrows)
    h2 = _tc_layer(h1, agg2, deg, W2_self, W2_neigh, b2, relu=True)
    (agg3,) = _sc_agg(h2, src, dst, zrows)
    h3 = _tc_layer(h2, agg3, deg, W3_self, W3_neigh, b3, relu=False)
    return h3[:N_NODES_C]
